# baseline (speedup 1.0000x reference)
import functools

import jax
import jax.numpy as jnp
from jax import lax
from jax.experimental import pallas as pl
from jax.experimental.pallas import tpu as pltpu
from jax.experimental.pallas import tpu_sc as plsc

NUM_CORES = 2
NUM_SUBCORES = 16
NUM_WORKERS = NUM_CORES * NUM_SUBCORES
REP = 8


def _make_sc_broadcast(batch, flat_len):
    rows_per_worker = batch // NUM_WORKERS
    n_out_dmas = rows_per_worker // REP
    mesh = plsc.VectorSubcoreMesh(core_axis_name="c", subcore_axis_name="s")

    @functools.partial(
        pl.kernel,
        mesh=mesh,
        out_type=jax.ShapeDtypeStruct((batch, flat_len), jnp.float32),
        scratch_types=[
            pltpu.VMEM((REP, flat_len), jnp.float32),
            pltpu.SemaphoreType.DMA,
            pltpu.SemaphoreType.DMA,
        ],
    )
    def sc_broadcast(flat_hbm, out_hbm, rep_v, in_sem, out_sem):
        wid = lax.axis_index("s") * NUM_CORES + lax.axis_index("c")
        base = wid * rows_per_worker
        fills = [
            pltpu.async_copy(flat_hbm, rep_v.at[j], in_sem) for j in range(REP)
        ]
        for f in fills:
            f.wait()
        stores = [
            pltpu.async_copy(
                rep_v, out_hbm.at[pl.ds(base + j * REP, REP)], out_sem
            )
            for j in range(n_out_dmas)
        ]
        for s in stores:
            s.wait()

    return sc_broadcast


def kernel(sequence, pos_table):
    batch, seq_len = sequence.shape
    hidden = pos_table.shape[1]
    flat = pos_table[:seq_len].reshape(seq_len * hidden)
    out = _make_sc_broadcast(batch, seq_len * hidden)(flat)
    return out.reshape(batch, seq_len, hidden)
